# baseline (speedup 1.0000x reference)
import functools

import jax
import jax.numpy as jnp
from jax import lax
from jax.experimental import pallas as pl
from jax.experimental.pallas import tpu as pltpu
from jax.experimental.pallas import tpu_sc as plsc

B = 16384
D = 16
H1 = 64
H2 = 32


@functools.cache
def _sc_gather():
    info = plsc.get_sparse_core_info()
    nw = info.num_cores * info.num_subcores
    bpw = B // nw
    mesh = plsc.VectorSubcoreMesh(core_axis_name="c", subcore_axis_name="s")

    @functools.partial(
        pl.kernel,
        out_type=(
            jax.ShapeDtypeStruct((B, D), jnp.float32),
            jax.ShapeDtypeStruct((B, D), jnp.float32),
        ),
        mesh=mesh,
        compiler_params=pltpu.CompilerParams(use_tc_tiling_on_sc=False),
        scratch_types=[
            pltpu.VMEM((bpw,), jnp.int32),
            pltpu.VMEM((bpw,), jnp.int32),
            pltpu.VMEM((bpw, D), jnp.float32),
            pltpu.VMEM((bpw, D), jnp.float32),
            pltpu.SemaphoreType.DMA,
            pltpu.SemaphoreType.DMA,
        ],
    )
    def gather_kernel(uid_hbm, iid_hbm, utab_hbm, itab_hbm, uout_hbm, iout_hbm,
                      uidx_v, iidx_v, urows_v, irows_v, usem, isem):
        wid = lax.axis_index("s") * info.num_cores + lax.axis_index("c")
        base = wid * bpw
        pltpu.sync_copy(uid_hbm.at[pl.ds(base, bpw)], uidx_v)
        pltpu.sync_copy(iid_hbm.at[pl.ds(base, bpw)], iidx_v)
        cu = pltpu.async_copy(utab_hbm.at[uidx_v], urows_v, usem)
        ci = pltpu.async_copy(itab_hbm.at[iidx_v], irows_v, isem)
        cu.wait()
        pltpu.sync_copy(urows_v, uout_hbm.at[pl.ds(base, bpw)])
        ci.wait()
        pltpu.sync_copy(irows_v, iout_hbm.at[pl.ds(base, bpw)])

    return gather_kernel


_BB = 2048


def _tc_body(u_ref, i_ref, w1u_ref, w1i_ref, b1_ref, w2_ref, b2_ref, w3_ref,
             c0_ref, out_ref):
    u = u_ref[...]
    it = i_ref[...]
    inter = jnp.sum(u * it, axis=1)
    h1 = jnp.dot(u, w1u_ref[...], preferred_element_type=jnp.float32)
    h1 = h1 + jnp.dot(it, w1i_ref[...], preferred_element_type=jnp.float32)
    h1 = jnp.maximum(h1 + b1_ref[...], 0.0)
    h2 = jnp.dot(h1, w2_ref[...], preferred_element_type=jnp.float32)
    h2 = jnp.maximum(h2 + b2_ref[...], 0.0)
    deep = jnp.sum(h2 * w3_ref[...], axis=1)
    out_ref[...] = inter + deep + c0_ref[0]


def _tc_mlp(u_emb, i_emb, w1u, w1i, b1, w2, b2, w3row, c0):
    rep = lambda shape: pl.BlockSpec(shape, lambda i: (0,) * len(shape))
    return pl.pallas_call(
        _tc_body,
        grid=(B // _BB,),
        in_specs=[
            pl.BlockSpec((_BB, D), lambda i: (i, 0)),
            pl.BlockSpec((_BB, D), lambda i: (i, 0)),
            rep((D, H1)),
            rep((D, H1)),
            rep((1, H1)),
            rep((H1, H2)),
            rep((1, H2)),
            rep((1, H2)),
            pl.BlockSpec(memory_space=pltpu.SMEM),
        ],
        out_specs=pl.BlockSpec((_BB,), lambda i: (i,)),
        out_shape=jax.ShapeDtypeStruct((B,), jnp.float32),
    )(u_emb, i_emb, w1u, w1i, b1, w2, b2, w3row, c0)


def kernel(user_id, item_id, user_table, item_table, fm_bias, W1, b1, W2, b2,
           W3, b3):
    uid = user_id.astype(jnp.int32)
    iid = item_id.astype(jnp.int32)
    u_emb, i_emb = _sc_gather()(uid, iid, user_table, item_table)
    c0 = fm_bias + b3
    return _tc_mlp(u_emb, i_emb, W1[:D], W1[D:], b1.reshape(1, H1), W2,
                   b2.reshape(1, H2), W3.reshape(1, H2), c0)
